# baseline (speedup 1.0000x reference)
import functools

import jax
import jax.numpy as jnp
from jax import lax
from jax.experimental import pallas as pl
from jax.experimental.pallas import tpu as pltpu
from jax.experimental.pallas import tpu_sc as plsc

B = 16384
F = 128
NC = 2
NS = 16
NW = NC * NS
RPW = B // NW
C = 128
NCH = RPW // C
NG = F // 16


def _sc_kernel(uidx_hbm, iidx_hbm, eu_hbm, ei_hbm, w_hbm, b16_hbm,
               prod_hbm, logit_hbm,
               idx_u, idx_i, urows, irows, prod, wv, bv, logit_v, acc_v,
               sem_g, sem_s, sem_p):
    wid = lax.axis_index("s") * NC + lax.axis_index("c")
    base = wid * RPW
    cbase = wid * NCH

    cp_w = pltpu.async_copy(w_hbm, wv, sem_p)
    cp_b = pltpu.async_copy(b16_hbm, bv, sem_p)
    cp_iu = pltpu.async_copy(uidx_hbm.at[pl.ds(cbase, NCH)], idx_u, sem_p)
    cp_ii = pltpu.async_copy(iidx_hbm.at[pl.ds(cbase, NCH)], idx_i, sem_p)
    cp_iu.wait()
    cp_ii.wait()
    cp_w.wait()
    cp_b.wait()
    wregs = [wv[pl.ds(j * 16, 16)] for j in range(NG)]
    breg = bv[...]
    lane = lax.iota(jnp.int32, 16)

    def issue_gathers(c, b):
        pltpu.async_copy(eu_hbm.at[idx_u.at[c]], urows.at[b], sem_g.at[b])
        pltpu.async_copy(ei_hbm.at[idx_i.at[c]], irows.at[b], sem_g.at[b])

    issue_gathers(0, 0)
    issue_gathers(1, 1)

    @pl.loop(0, NCH)
    def chunk_loop(c):
        b = lax.rem(c, 2)
        @pl.when(c >= 2)
        def _():
            pltpu.make_async_copy(
                prod.at[b], prod_hbm.at[pl.ds(base, C)], sem_s.at[b]).wait()

        for _ in range(2):
            pltpu.make_async_copy(
                eu_hbm.at[pl.ds(0, C)], urows.at[b], sem_g.at[b]).wait()

        ub, ib, pb = urows.at[b], irows.at[b], prod.at[b]

        def grp_body(g, _, ub=ub, ib=ib, pb=pb, c=c):
            for rr in range(16):
                r = g * 16 + rr
                acc = breg
                for j in range(NG):
                    u = ub[r, pl.ds(j * 16, 16)]
                    i = ib[r, pl.ds(j * 16, 16)]
                    p = u * i
                    pb[r, pl.ds(j * 16, 16)] = p
                    acc = acc + p * wregs[j]
                acc_v[0, rr] = acc
            lvec = plsc.load_gather(
                acc_v, [jnp.zeros((16,), jnp.int32), lane,
                        jnp.zeros((16,), jnp.int32)])
            for j in range(1, 16):
                lvec = lvec + plsc.load_gather(
                    acc_v, [jnp.zeros((16,), jnp.int32), lane,
                            jnp.full((16,), j, jnp.int32)])
            logit_v[pl.ds(c * C + g * 16, 16)] = lvec
            return _

        lax.fori_loop(0, C // 16, grp_body, None)

        @pl.when(c + 2 < NCH)
        def _():
            issue_gathers(c + 2, b)

        pltpu.async_copy(
            prod.at[b], prod_hbm.at[pl.ds(base + c * C, C)], sem_s.at[b])

    pltpu.sync_copy(logit_v, logit_hbm.at[pl.ds(base, RPW)])
    for b in range(2):
        pltpu.make_async_copy(
            prod.at[b], prod_hbm.at[pl.ds(base, C)], sem_s.at[b]).wait()


@jax.jit
def _run(uidx2d, iidx2d, embed_user, embed_item, w_flat, b16):
    mesh = plsc.VectorSubcoreMesh(core_axis_name="c", subcore_axis_name="s",
                                  num_cores=NC, num_subcores=NS)
    f = functools.partial(
        pl.kernel,
        out_type=[jax.ShapeDtypeStruct((B, F), jnp.float32),
                  jax.ShapeDtypeStruct((B,), jnp.float32)],
        mesh=mesh,
        compiler_params=pltpu.CompilerParams(needs_layout_passes=False),
        scratch_types=[
            pltpu.VMEM((NCH, C), jnp.int32),
            pltpu.VMEM((NCH, C), jnp.int32),
            pltpu.VMEM((2, C, F), jnp.float32),
            pltpu.VMEM((2, C, F), jnp.float32),
            pltpu.VMEM((2, C, F), jnp.float32),
            pltpu.VMEM((F,), jnp.float32),
            pltpu.VMEM((16,), jnp.float32),
            pltpu.VMEM((RPW,), jnp.float32),
            pltpu.VMEM((C // 16, 16, 16), jnp.float32),
            pltpu.SemaphoreType.DMA((2,)),
            pltpu.SemaphoreType.DMA((2,)),
            pltpu.SemaphoreType.DMA,
        ],
    )(_sc_kernel)
    return f(uidx2d, iidx2d, embed_user, embed_item, w_flat, b16)


def kernel(user_idx, item_idx, embed_user, embed_item, W, b):
    uidx2d = user_idx.reshape(B // C, C)
    iidx2d = item_idx.reshape(B // C, C)
    w_flat = W[:, 0]
    b16 = jnp.concatenate([b, jnp.zeros((15,), jnp.float32)])
    pred_vector, logit = _run(uidx2d, iidx2d, embed_user, embed_item,
                              w_flat, b16)
    return (pred_vector, logit)


# speedup vs baseline: 1.3600x; 1.3600x over previous
import functools

import jax
import jax.numpy as jnp
from jax import lax
from jax.experimental import pallas as pl
from jax.experimental.pallas import tpu as pltpu
from jax.experimental.pallas import tpu_sc as plsc

B = 16384
F = 128
NC = 2
NS = 16
NW = NC * NS
RPW = B // NW
C = 64
NCH = RPW // C
NG = F // 16


def _sc_kernel(uidx_hbm, iidx_hbm, eu_hbm, ei_hbm, w_hbm, b16_hbm,
               prod_hbm, logit_hbm,
               idx_u, idx_i, urows, irows, prod, wv, bv, logit_v, acc_v,
               sem_g0, sem_g1, sem_s0, sem_s1):
    wid = lax.axis_index("s") * NC + lax.axis_index("c")
    base = wid * RPW
    cbase = wid * NCH

    cp_w = pltpu.async_copy(w_hbm, wv, sem_s0)
    cp_b = pltpu.async_copy(b16_hbm, bv, sem_s0)
    cp_iu = pltpu.async_copy(uidx_hbm.at[pl.ds(cbase, NCH)], idx_u, sem_s1)
    cp_ii = pltpu.async_copy(iidx_hbm.at[pl.ds(cbase, NCH)], idx_i, sem_s1)
    cp_iu.wait()
    cp_ii.wait()
    cp_w.wait()
    cp_b.wait()
    wregs = [wv[pl.ds(j * 16, 16)] for j in range(NG)]
    breg = bv[...]
    lane = lax.iota(jnp.int32, 16)
    sem_g = [sem_g0, sem_g1]
    sem_s = [sem_s0, sem_s1]

    def issue_gathers(c, b):
        ub, ib = (urows.at[b], irows.at[b])
        pltpu.async_copy(eu_hbm.at[idx_u.at[c]], ub, sem_g[b])
        pltpu.async_copy(ei_hbm.at[idx_i.at[c]], ib, sem_g[b])

    issue_gathers(0, 0)
    issue_gathers(1, 1)

    @pl.loop(0, NCH, step=2)
    def chunk_pair(cp):
        for b in range(2):
            c = cp + b
            @pl.when(cp > 0)
            def _():
                pltpu.make_async_copy(
                    prod.at[b], prod_hbm.at[pl.ds(base, C)], sem_s[b]).wait()

            for _ in range(2):
                pltpu.make_async_copy(
                    eu_hbm.at[pl.ds(0, C)], urows.at[b], sem_g[b]).wait()

            ub, ib, pb = urows.at[b], irows.at[b], prod.at[b]

            def grp_body(g, _, ub=ub, ib=ib, pb=pb, c=c):
                for rr in range(16):
                    r = g * 16 + rr
                    acc = breg
                    for j in range(NG):
                        u = ub[r, pl.ds(j * 16, 16)]
                        i = ib[r, pl.ds(j * 16, 16)]
                        p = u * i
                        pb[r, pl.ds(j * 16, 16)] = p
                        acc = acc + p * wregs[j]
                    acc_v[0, rr] = acc
                lvec = plsc.load_gather(
                    acc_v, [jnp.zeros((16,), jnp.int32), lane,
                            jnp.zeros((16,), jnp.int32)])
                for j in range(1, 16):
                    lvec = lvec + plsc.load_gather(
                        acc_v, [jnp.zeros((16,), jnp.int32), lane,
                                jnp.full((16,), j, jnp.int32)])
                logit_v[pl.ds(c * C + g * 16, 16)] = lvec
                return _

            lax.fori_loop(0, C // 16, grp_body, None)

            @pl.when(cp + 2 < NCH)
            def _():
                issue_gathers(c + 2, b)

            pltpu.async_copy(
                prod.at[b], prod_hbm.at[pl.ds(base + c * C, C)], sem_s[b])

    pltpu.sync_copy(logit_v, logit_hbm.at[pl.ds(base, RPW)])
    for b in range(2):
        pltpu.make_async_copy(
            prod.at[b], prod_hbm.at[pl.ds(base, C)], sem_s[b]).wait()


@jax.jit
def _run(uidx2d, iidx2d, embed_user, embed_item, w_flat, b16):
    mesh = plsc.VectorSubcoreMesh(core_axis_name="c", subcore_axis_name="s",
                                  num_cores=NC, num_subcores=NS)
    f = functools.partial(
        pl.kernel,
        out_type=[jax.ShapeDtypeStruct((B, F), jnp.float32),
                  jax.ShapeDtypeStruct((B,), jnp.float32)],
        mesh=mesh,
        compiler_params=pltpu.CompilerParams(needs_layout_passes=False),
        scratch_types=[
            pltpu.VMEM((NCH, C), jnp.int32),
            pltpu.VMEM((NCH, C), jnp.int32),
            pltpu.VMEM((2, C, F), jnp.float32),
            pltpu.VMEM((2, C, F), jnp.float32),
            pltpu.VMEM((2, C, F), jnp.float32),
            pltpu.VMEM((F,), jnp.float32),
            pltpu.VMEM((16,), jnp.float32),
            pltpu.VMEM((RPW,), jnp.float32),
            pltpu.VMEM((C // 16, 16, 16), jnp.float32),
            pltpu.SemaphoreType.DMA,
            pltpu.SemaphoreType.DMA,
            pltpu.SemaphoreType.DMA,
            pltpu.SemaphoreType.DMA,
        ],
    )(_sc_kernel)
    return f(uidx2d, iidx2d, embed_user, embed_item, w_flat, b16)


def kernel(user_idx, item_idx, embed_user, embed_item, W, b):
    uidx2d = user_idx.reshape(B // C, C)
    iidx2d = item_idx.reshape(B // C, C)
    w_flat = W[:, 0]
    b16 = jnp.concatenate([b, jnp.zeros((15,), jnp.float32)])
    pred_vector, logit = _run(uidx2d, iidx2d, embed_user, embed_item,
                              w_flat, b16)
    return (pred_vector, logit)
